# Initial kernel scaffold; baseline (speedup 1.0000x reference)
import jax
import jax.numpy as jnp
from jax.experimental import pallas as pl


def kernel(block_outputs, pos_edge_index, neg_edge_index):
    raise NotImplementedError("write your pallas kernel here")


# speedup vs baseline: 1.1390x; 1.1390x over previous
import functools

import jax
import jax.numpy as jnp
from jax import lax
from jax.experimental import pallas as pl
from jax.experimental.pallas import tpu as pltpu
from jax.experimental.pallas import tpu_sc as plsc

N = 10000
D = 128
E_POS = 320000
E_NEG = 320000
E = E_POS + E_NEG

NC = 2
NS = 16
NW = NC * NS
L = 16

EDGES_PER_TILE = E // NW
CHUNK = 400
N_CHUNKS = EDGES_PER_TILE // CHUNK
GROUPS = CHUNK // L


def _sc_scores_body(h_hbm, src_hbm, dst_hbm, out_hbm,
                    idx_s_v, idx_d_v, rows_s_v, rows_d_v, scores_v, sem):
    wid = lax.axis_index("s") * NC + lax.axis_index("c")
    tile_base = wid * EDGES_PER_TILE

    def chunk_body(c, carry):
        off = tile_base + c * CHUNK
        pltpu.sync_copy(src_hbm.at[pl.ds(off, CHUNK)], idx_s_v)
        pltpu.sync_copy(dst_hbm.at[pl.ds(off, CHUNK)], idx_d_v)
        pltpu.async_copy(h_hbm.at[idx_s_v], rows_s_v, sem).wait()
        pltpu.async_copy(h_hbm.at[idx_d_v], rows_d_v, sem).wait()

        def group_body(g, carry2):
            row_idx = g * L + lax.iota(jnp.int32, L)

            def k_body(k, acc):
                col = jnp.full((L,), k, dtype=jnp.int32)
                s = plsc.load_gather(rows_s_v, [row_idx, col])
                d = plsc.load_gather(rows_d_v, [row_idx, col])
                return acc + s * d

            acc = lax.fori_loop(0, D, k_body, jnp.zeros((L,), jnp.float32))
            scores_v[pl.ds(g * L, L)] = acc
            return carry2

        lax.fori_loop(0, GROUPS, group_body, 0)
        pltpu.sync_copy(scores_v, out_hbm.at[pl.ds(off, CHUNK)])
        return carry

    lax.fori_loop(0, N_CHUNKS, chunk_body, 0)


@jax.jit
def _sc_scores(h, src, dst):
    mesh = plsc.VectorSubcoreMesh(core_axis_name="c", subcore_axis_name="s")
    fn = pl.kernel(
        _sc_scores_body,
        mesh=mesh,
        compiler_params=pltpu.CompilerParams(needs_layout_passes=False),
        out_type=jax.ShapeDtypeStruct((E,), jnp.float32),
        scratch_types=[
            pltpu.VMEM((CHUNK,), jnp.int32),
            pltpu.VMEM((CHUNK,), jnp.int32),
            pltpu.VMEM((CHUNK, D), jnp.float32),
            pltpu.VMEM((CHUNK, D), jnp.float32),
            pltpu.VMEM((CHUNK,), jnp.float32),
            pltpu.SemaphoreType.DMA,
        ],
    )
    return fn(h, src, dst)


def _tc_loss_body(scores_ref, out_ref):
    s = scores_ref[...]
    rows = lax.broadcasted_iota(jnp.int32, s.shape, 0)
    label = (rows < (E_POS // 128)).astype(jnp.float32)
    terms = jnp.maximum(s, 0.0) - s * label + jnp.log1p(jnp.exp(-jnp.abs(s)))
    out_ref[0, 0] = jnp.sum(terms) * (1.0 / E)


@jax.jit
def _tc_loss(scores2d):
    return pl.pallas_call(
        _tc_loss_body,
        out_shape=jax.ShapeDtypeStruct((1, 1), jnp.float32),
        out_specs=pl.BlockSpec(memory_space=pltpu.SMEM),
    )(scores2d)


def kernel(block_outputs, pos_edge_index, neg_edge_index):
    src = jnp.concatenate([pos_edge_index[0], neg_edge_index[0]])
    dst = jnp.concatenate([pos_edge_index[1], neg_edge_index[1]])
    scores = _sc_scores(block_outputs, src, dst)
    loss = _tc_loss(scores.reshape(E // 128, 128))
    return loss[0, 0]


# speedup vs baseline: 1.2856x; 1.1287x over previous
import jax
import jax.numpy as jnp
from jax import lax
from jax.experimental import pallas as pl
from jax.experimental.pallas import tpu as pltpu
from jax.experimental.pallas import tpu_sc as plsc

N = 10000
D = 128
E_POS = 320000
E_NEG = 320000
E = E_POS + E_NEG

NC = 2
NS = 16
NW = NC * NS
L = 16

EDGES_PER_TILE = E // NW
CHUNK = 80
N_CHUNKS = EDGES_PER_TILE // CHUNK
PAIRS = N_CHUNKS // 2
GROUPS = CHUNK // L


def _sc_scores_body(h_hbm, src_hbm, dst_hbm, out_hbm,
                    idx_s_v, idx_d_v,
                    rows_s0, rows_d0, rows_s1, rows_d1,
                    scores_v,
                    sem_s0, sem_d0, sem_s1, sem_d1):
    wid = lax.axis_index("s") * NC + lax.axis_index("c")
    tile_base = wid * EDGES_PER_TILE

    pltpu.sync_copy(src_hbm.at[pl.ds(tile_base, EDGES_PER_TILE)], idx_s_v)
    pltpu.sync_copy(dst_hbm.at[pl.ds(tile_base, EDGES_PER_TILE)], idx_d_v)

    def issue(c, rows_s, rows_d, sem_s, sem_d):
        off = c * CHUNK
        pltpu.async_copy(h_hbm.at[idx_s_v.at[pl.ds(off, CHUNK)]], rows_s, sem_s)
        pltpu.async_copy(h_hbm.at[idx_d_v.at[pl.ds(off, CHUNK)]], rows_d, sem_d)

    def wait(c, rows_s, rows_d, sem_s, sem_d):
        off = c * CHUNK
        pltpu.make_async_copy(
            h_hbm.at[idx_s_v.at[pl.ds(off, CHUNK)]], rows_s, sem_s).wait()
        pltpu.make_async_copy(
            h_hbm.at[idx_d_v.at[pl.ds(off, CHUNK)]], rows_d, sem_d).wait()

    def compute(c, rows_s, rows_d):
        score_base = c * CHUNK

        def group_body(g, carry):
            row = g * L + lax.iota(jnp.int32, L)
            acc = jnp.zeros((L,), jnp.float32)
            for k in range(D):
                col = jnp.full((L,), k, dtype=jnp.int32)
                s = plsc.load_gather(rows_s, [row, col])
                d = plsc.load_gather(rows_d, [row, col])
                acc = acc + s * d
            scores_v[pl.ds(score_base + g * L, L)] = acc
            return carry

        lax.fori_loop(0, GROUPS, group_body, 0)

    issue(0, rows_s0, rows_d0, sem_s0, sem_d0)

    def pair_body(co, carry):
        a = 2 * co
        b = a + 1
        issue(b, rows_s1, rows_d1, sem_s1, sem_d1)
        wait(a, rows_s0, rows_d0, sem_s0, sem_d0)
        compute(a, rows_s0, rows_d0)

        @pl.when(co < PAIRS - 1)
        def _():
            issue(a + 2, rows_s0, rows_d0, sem_s0, sem_d0)

        wait(b, rows_s1, rows_d1, sem_s1, sem_d1)
        compute(b, rows_s1, rows_d1)
        return carry

    lax.fori_loop(0, PAIRS, pair_body, 0)
    pltpu.sync_copy(scores_v, out_hbm.at[pl.ds(tile_base, EDGES_PER_TILE)])


@jax.jit
def _sc_scores(h, src, dst):
    mesh = plsc.VectorSubcoreMesh(core_axis_name="c", subcore_axis_name="s")
    fn = pl.kernel(
        _sc_scores_body,
        mesh=mesh,
        compiler_params=pltpu.CompilerParams(needs_layout_passes=False),
        out_type=jax.ShapeDtypeStruct((E,), jnp.float32),
        scratch_types=[
            pltpu.VMEM((EDGES_PER_TILE,), jnp.int32),
            pltpu.VMEM((EDGES_PER_TILE,), jnp.int32),
            pltpu.VMEM((CHUNK, D), jnp.float32),
            pltpu.VMEM((CHUNK, D), jnp.float32),
            pltpu.VMEM((CHUNK, D), jnp.float32),
            pltpu.VMEM((CHUNK, D), jnp.float32),
            pltpu.VMEM((EDGES_PER_TILE,), jnp.float32),
            pltpu.SemaphoreType.DMA,
            pltpu.SemaphoreType.DMA,
            pltpu.SemaphoreType.DMA,
            pltpu.SemaphoreType.DMA,
        ],
    )
    return fn(h, src, dst)


def _tc_loss_body(scores_ref, out_ref):
    s = scores_ref[...]
    rows = lax.broadcasted_iota(jnp.int32, s.shape, 0)
    label = (rows < (E_POS // 128)).astype(jnp.float32)
    terms = jnp.maximum(s, 0.0) - s * label + jnp.log1p(jnp.exp(-jnp.abs(s)))
    out_ref[0, 0] = jnp.sum(terms) * (1.0 / E)


@jax.jit
def _tc_loss(scores2d):
    return pl.pallas_call(
        _tc_loss_body,
        out_shape=jax.ShapeDtypeStruct((1, 1), jnp.float32),
        out_specs=pl.BlockSpec(memory_space=pltpu.SMEM),
    )(scores2d)


def kernel(block_outputs, pos_edge_index, neg_edge_index):
    src = jnp.concatenate([pos_edge_index[0], neg_edge_index[0]])
    dst = jnp.concatenate([pos_edge_index[1], neg_edge_index[1]])
    scores = _sc_scores(block_outputs, src, dst)
    loss = _tc_loss(scores.reshape(E // 128, 128))
    return loss[0, 0]


# speedup vs baseline: 4.0032x; 3.1139x over previous
import jax
import jax.numpy as jnp
from jax import lax
from jax.experimental import pallas as pl
from jax.experimental.pallas import tpu as pltpu
from jax.experimental.pallas import tpu_sc as plsc

N = 10000
D = 128
E_POS = 320000
E_NEG = 320000
E = E_POS + E_NEG

NC = 2
NS = 16
NW = NC * NS
L = 16

EDGES_PER_TILE = E // NW
CHUNK = 80
N_CHUNKS = EDGES_PER_TILE // CHUNK
PAIRS = N_CHUNKS // 2
GROUPS = CHUNK // L


def _sc_scores_body(h_hbm, src_hbm, dst_hbm, out_hbm,
                    idx_s_v, idx_d_v,
                    rows_s0, rows_d0, rows_s1, rows_d1,
                    scores_v,
                    sem_s0, sem_d0, sem_s1, sem_d1):
    wid = lax.axis_index("s") * NC + lax.axis_index("c")
    tile_base = wid * EDGES_PER_TILE

    pltpu.sync_copy(src_hbm.at[pl.ds(tile_base, EDGES_PER_TILE)], idx_s_v)
    pltpu.sync_copy(dst_hbm.at[pl.ds(tile_base, EDGES_PER_TILE)], idx_d_v)

    def issue(c, rows_s, rows_d, sem_s, sem_d):
        off = c * CHUNK
        pltpu.async_copy(h_hbm.at[idx_s_v.at[pl.ds(off, CHUNK)]], rows_s, sem_s)
        pltpu.async_copy(h_hbm.at[idx_d_v.at[pl.ds(off, CHUNK)]], rows_d, sem_d)

    def wait(c, rows_s, rows_d, sem_s, sem_d):
        off = c * CHUNK
        pltpu.make_async_copy(
            h_hbm.at[idx_s_v.at[pl.ds(off, CHUNK)]], rows_s, sem_s).wait()
        pltpu.make_async_copy(
            h_hbm.at[idx_d_v.at[pl.ds(off, CHUNK)]], rows_d, sem_d).wait()

    def compute(c, rows_s, rows_d):
        score_base = c * CHUNK

        lane = lax.iota(jnp.int32, L)

        def group_body(g, carry):
            ebase = g * L
            out = jnp.zeros((L,), jnp.float32)
            for e in range(L):
                ei = ebase + e
                acc = rows_s[ei, pl.ds(0, L)] * rows_d[ei, pl.ds(0, L)]
                for kb in range(1, D // L):
                    acc = acc + (rows_s[ei, pl.ds(kb * L, L)]
                                 * rows_d[ei, pl.ds(kb * L, L)])
                out = jnp.where(lane == e, jnp.sum(acc), out)
            scores_v[pl.ds(score_base + ebase, L)] = out
            return carry

        lax.fori_loop(0, GROUPS, group_body, 0)

    issue(0, rows_s0, rows_d0, sem_s0, sem_d0)

    def pair_body(co, carry):
        a = 2 * co
        b = a + 1
        issue(b, rows_s1, rows_d1, sem_s1, sem_d1)
        wait(a, rows_s0, rows_d0, sem_s0, sem_d0)
        compute(a, rows_s0, rows_d0)

        @pl.when(co < PAIRS - 1)
        def _():
            issue(a + 2, rows_s0, rows_d0, sem_s0, sem_d0)

        wait(b, rows_s1, rows_d1, sem_s1, sem_d1)
        compute(b, rows_s1, rows_d1)
        return carry

    lax.fori_loop(0, PAIRS, pair_body, 0)
    pltpu.sync_copy(scores_v, out_hbm.at[pl.ds(tile_base, EDGES_PER_TILE)])


@jax.jit
def _sc_scores(h, src, dst):
    mesh = plsc.VectorSubcoreMesh(core_axis_name="c", subcore_axis_name="s")
    fn = pl.kernel(
        _sc_scores_body,
        mesh=mesh,
        compiler_params=pltpu.CompilerParams(needs_layout_passes=False),
        out_type=jax.ShapeDtypeStruct((E,), jnp.float32),
        scratch_types=[
            pltpu.VMEM((EDGES_PER_TILE,), jnp.int32),
            pltpu.VMEM((EDGES_PER_TILE,), jnp.int32),
            pltpu.VMEM((CHUNK, D), jnp.float32),
            pltpu.VMEM((CHUNK, D), jnp.float32),
            pltpu.VMEM((CHUNK, D), jnp.float32),
            pltpu.VMEM((CHUNK, D), jnp.float32),
            pltpu.VMEM((EDGES_PER_TILE,), jnp.float32),
            pltpu.SemaphoreType.DMA,
            pltpu.SemaphoreType.DMA,
            pltpu.SemaphoreType.DMA,
            pltpu.SemaphoreType.DMA,
        ],
    )
    return fn(h, src, dst)


def _tc_loss_body(scores_ref, out_ref):
    s = scores_ref[...]
    rows = lax.broadcasted_iota(jnp.int32, s.shape, 0)
    label = (rows < (E_POS // 128)).astype(jnp.float32)
    terms = jnp.maximum(s, 0.0) - s * label + jnp.log1p(jnp.exp(-jnp.abs(s)))
    out_ref[0, 0] = jnp.sum(terms) * (1.0 / E)


@jax.jit
def _tc_loss(scores2d):
    return pl.pallas_call(
        _tc_loss_body,
        out_shape=jax.ShapeDtypeStruct((1, 1), jnp.float32),
        out_specs=pl.BlockSpec(memory_space=pltpu.SMEM),
    )(scores2d)


def kernel(block_outputs, pos_edge_index, neg_edge_index):
    src = jnp.concatenate([pos_edge_index[0], neg_edge_index[0]])
    dst = jnp.concatenate([pos_edge_index[1], neg_edge_index[1]])
    scores = _sc_scores(block_outputs, src, dst)
    loss = _tc_loss(scores.reshape(E // 128, 128))
    return loss[0, 0]


# speedup vs baseline: 4.6241x; 1.1551x over previous
import jax
import jax.numpy as jnp
from jax import lax
from jax.experimental import pallas as pl
from jax.experimental.pallas import tpu as pltpu
from jax.experimental.pallas import tpu_sc as plsc

N = 10000
D = 128
E_POS = 320000
E_NEG = 320000
E = E_POS + E_NEG

NC = 2
NS = 16
NW = NC * NS
L = 16

EDGES_PER_TILE = E // NW
CHUNK = 80
N_CHUNKS = EDGES_PER_TILE // CHUNK
PAIRS = N_CHUNKS // 2
GROUPS = CHUNK // L


def _sc_scores_body(h_hbm, src_hbm, dst_hbm, out_hbm,
                    idx_s_v, idx_d_v,
                    rows_s0, rows_d0, rows_s1, rows_d1,
                    scores_v,
                    sem_s0, sem_d0, sem_s1, sem_d1):
    wid = lax.axis_index("s") * NC + lax.axis_index("c")
    tile_base = wid * EDGES_PER_TILE

    pltpu.sync_copy(src_hbm.at[pl.ds(tile_base, EDGES_PER_TILE)], idx_s_v)
    pltpu.sync_copy(dst_hbm.at[pl.ds(tile_base, EDGES_PER_TILE)], idx_d_v)

    def issue(c, rows_s, rows_d, sem_s, sem_d):
        off = c * CHUNK
        pltpu.async_copy(h_hbm.at[idx_s_v.at[pl.ds(off, CHUNK)]], rows_s, sem_s)
        pltpu.async_copy(h_hbm.at[idx_d_v.at[pl.ds(off, CHUNK)]], rows_d, sem_d)

    def wait(c, rows_s, rows_d, sem_s, sem_d):
        off = c * CHUNK
        pltpu.make_async_copy(
            h_hbm.at[idx_s_v.at[pl.ds(off, CHUNK)]], rows_s, sem_s).wait()
        pltpu.make_async_copy(
            h_hbm.at[idx_d_v.at[pl.ds(off, CHUNK)]], rows_d, sem_d).wait()

    def compute(c, rows_s, rows_d):
        score_base = c * CHUNK

        lane = lax.iota(jnp.int32, L)

        def group_body(g, carry):
            row = g * L + lane
            acc = jnp.zeros((L,), jnp.float32)
            col = lane
            for _ in range(D):
                s = plsc.load_gather(rows_s, [row, col])
                d = plsc.load_gather(rows_d, [row, col])
                acc = acc + s * d
                col = (col + 1) & (D - 1)
            scores_v[pl.ds(score_base + g * L, L)] = acc
            return carry

        lax.fori_loop(0, GROUPS, group_body, 0)

    issue(0, rows_s0, rows_d0, sem_s0, sem_d0)

    def pair_body(co, carry):
        a = 2 * co
        b = a + 1
        issue(b, rows_s1, rows_d1, sem_s1, sem_d1)
        wait(a, rows_s0, rows_d0, sem_s0, sem_d0)
        compute(a, rows_s0, rows_d0)

        @pl.when(co < PAIRS - 1)
        def _():
            issue(a + 2, rows_s0, rows_d0, sem_s0, sem_d0)

        wait(b, rows_s1, rows_d1, sem_s1, sem_d1)
        compute(b, rows_s1, rows_d1)
        return carry

    lax.fori_loop(0, PAIRS, pair_body, 0)
    pltpu.sync_copy(scores_v, out_hbm.at[pl.ds(tile_base, EDGES_PER_TILE)])


@jax.jit
def _sc_scores(h, src, dst):
    mesh = plsc.VectorSubcoreMesh(core_axis_name="c", subcore_axis_name="s")
    fn = pl.kernel(
        _sc_scores_body,
        mesh=mesh,
        compiler_params=pltpu.CompilerParams(needs_layout_passes=False),
        out_type=jax.ShapeDtypeStruct((E,), jnp.float32),
        scratch_types=[
            pltpu.VMEM((EDGES_PER_TILE,), jnp.int32),
            pltpu.VMEM((EDGES_PER_TILE,), jnp.int32),
            pltpu.VMEM((CHUNK, D), jnp.float32),
            pltpu.VMEM((CHUNK, D), jnp.float32),
            pltpu.VMEM((CHUNK, D), jnp.float32),
            pltpu.VMEM((CHUNK, D), jnp.float32),
            pltpu.VMEM((EDGES_PER_TILE,), jnp.float32),
            pltpu.SemaphoreType.DMA,
            pltpu.SemaphoreType.DMA,
            pltpu.SemaphoreType.DMA,
            pltpu.SemaphoreType.DMA,
        ],
    )
    return fn(h, src, dst)


def _tc_loss_body(scores_ref, out_ref):
    s = scores_ref[...]
    rows = lax.broadcasted_iota(jnp.int32, s.shape, 0)
    label = (rows < (E_POS // 128)).astype(jnp.float32)
    terms = jnp.maximum(s, 0.0) - s * label + jnp.log1p(jnp.exp(-jnp.abs(s)))
    out_ref[0, 0] = jnp.sum(terms) * (1.0 / E)


@jax.jit
def _tc_loss(scores2d):
    return pl.pallas_call(
        _tc_loss_body,
        out_shape=jax.ShapeDtypeStruct((1, 1), jnp.float32),
        out_specs=pl.BlockSpec(memory_space=pltpu.SMEM),
    )(scores2d)


def kernel(block_outputs, pos_edge_index, neg_edge_index):
    src = jnp.concatenate([pos_edge_index[0], neg_edge_index[0]])
    dst = jnp.concatenate([pos_edge_index[1], neg_edge_index[1]])
    scores = _sc_scores(block_outputs, src, dst)
    loss = _tc_loss(scores.reshape(E // 128, 128))
    return loss[0, 0]


# speedup vs baseline: 9.1850x; 1.9863x over previous
import jax
import jax.numpy as jnp
from jax import lax
from jax.experimental import pallas as pl
from jax.experimental.pallas import tpu as pltpu
from jax.experimental.pallas import tpu_sc as plsc

N = 10000
D = 128
E_POS = 320000
E_NEG = 320000
E = E_POS + E_NEG

NC = 2
NS = 16
NW = NC * NS
L = 16

EDGES_PER_TILE = E // NW
CHUNK = 80
N_CHUNKS = EDGES_PER_TILE // CHUNK
PAIRS = N_CHUNKS // 2
GROUPS = CHUNK // L
P1_UNROLL = 4


def _sc_scores_body(h_hbm, src_hbm, dst_hbm, out_hbm,
                    idx_s_v, idx_d_v,
                    rows_s0, rows_d0, rows_s1, rows_d1,
                    partials_v, scores_v,
                    sem_s0, sem_d0, sem_s1, sem_d1):
    wid = lax.axis_index("s") * NC + lax.axis_index("c")
    tile_base = wid * EDGES_PER_TILE

    pltpu.sync_copy(src_hbm.at[pl.ds(tile_base, EDGES_PER_TILE)], idx_s_v)
    pltpu.sync_copy(dst_hbm.at[pl.ds(tile_base, EDGES_PER_TILE)], idx_d_v)

    def issue(c, rows_s, rows_d, sem_s, sem_d):
        off = c * CHUNK
        pltpu.async_copy(h_hbm.at[idx_s_v.at[pl.ds(off, CHUNK)]], rows_s, sem_s)
        pltpu.async_copy(h_hbm.at[idx_d_v.at[pl.ds(off, CHUNK)]], rows_d, sem_d)

    def wait(c, rows_s, rows_d, sem_s, sem_d):
        off = c * CHUNK
        pltpu.make_async_copy(
            h_hbm.at[idx_s_v.at[pl.ds(off, CHUNK)]], rows_s, sem_s).wait()
        pltpu.make_async_copy(
            h_hbm.at[idx_d_v.at[pl.ds(off, CHUNK)]], rows_d, sem_d).wait()

    def compute(c, rows_s, rows_d):
        score_base = c * CHUNK

        lane = lax.iota(jnp.int32, L)

        def p1_body(i, carry):
            for u in range(P1_UNROLL):
                ei = i * P1_UNROLL + u
                m0 = (rows_s[ei, pl.ds(0 * L, L)] * rows_d[ei, pl.ds(0 * L, L)]
                      + rows_s[ei, pl.ds(1 * L, L)] * rows_d[ei, pl.ds(1 * L, L)])
                m1 = (rows_s[ei, pl.ds(2 * L, L)] * rows_d[ei, pl.ds(2 * L, L)]
                      + rows_s[ei, pl.ds(3 * L, L)] * rows_d[ei, pl.ds(3 * L, L)])
                m2 = (rows_s[ei, pl.ds(4 * L, L)] * rows_d[ei, pl.ds(4 * L, L)]
                      + rows_s[ei, pl.ds(5 * L, L)] * rows_d[ei, pl.ds(5 * L, L)])
                m3 = (rows_s[ei, pl.ds(6 * L, L)] * rows_d[ei, pl.ds(6 * L, L)]
                      + rows_s[ei, pl.ds(7 * L, L)] * rows_d[ei, pl.ds(7 * L, L)])
                partials_v[ei, pl.ds(0, L)] = (m0 + m1) + (m2 + m3)
            return carry

        lax.fori_loop(0, CHUNK // P1_UNROLL, p1_body, 0)

        def p2_body(g, carry):
            row = g * L + lane
            acc = jnp.zeros((L,), jnp.float32)
            col = lane
            for _ in range(L):
                acc = acc + plsc.load_gather(partials_v, [row, col])
                col = (col + 1) & (L - 1)
            scores_v[pl.ds(score_base + g * L, L)] = acc
            return carry

        lax.fori_loop(0, GROUPS, p2_body, 0)

    issue(0, rows_s0, rows_d0, sem_s0, sem_d0)

    def pair_body(co, carry):
        a = 2 * co
        b = a + 1
        issue(b, rows_s1, rows_d1, sem_s1, sem_d1)
        wait(a, rows_s0, rows_d0, sem_s0, sem_d0)
        compute(a, rows_s0, rows_d0)

        @pl.when(co < PAIRS - 1)
        def _():
            issue(a + 2, rows_s0, rows_d0, sem_s0, sem_d0)

        wait(b, rows_s1, rows_d1, sem_s1, sem_d1)
        compute(b, rows_s1, rows_d1)
        return carry

    lax.fori_loop(0, PAIRS, pair_body, 0)
    pltpu.sync_copy(scores_v, out_hbm.at[pl.ds(tile_base, EDGES_PER_TILE)])


@jax.jit
def _sc_scores(h, src, dst):
    mesh = plsc.VectorSubcoreMesh(core_axis_name="c", subcore_axis_name="s")
    fn = pl.kernel(
        _sc_scores_body,
        mesh=mesh,
        compiler_params=pltpu.CompilerParams(needs_layout_passes=False),
        out_type=jax.ShapeDtypeStruct((E,), jnp.float32),
        scratch_types=[
            pltpu.VMEM((EDGES_PER_TILE,), jnp.int32),
            pltpu.VMEM((EDGES_PER_TILE,), jnp.int32),
            pltpu.VMEM((CHUNK, D), jnp.float32),
            pltpu.VMEM((CHUNK, D), jnp.float32),
            pltpu.VMEM((CHUNK, D), jnp.float32),
            pltpu.VMEM((CHUNK, D), jnp.float32),
            pltpu.VMEM((CHUNK, L), jnp.float32),
            pltpu.VMEM((EDGES_PER_TILE,), jnp.float32),
            pltpu.SemaphoreType.DMA,
            pltpu.SemaphoreType.DMA,
            pltpu.SemaphoreType.DMA,
            pltpu.SemaphoreType.DMA,
        ],
    )
    return fn(h, src, dst)


def _tc_loss_body(scores_ref, out_ref):
    s = scores_ref[...]
    rows = lax.broadcasted_iota(jnp.int32, s.shape, 0)
    label = (rows < (E_POS // 128)).astype(jnp.float32)
    terms = jnp.maximum(s, 0.0) - s * label + jnp.log1p(jnp.exp(-jnp.abs(s)))
    out_ref[0, 0] = jnp.sum(terms) * (1.0 / E)


@jax.jit
def _tc_loss(scores2d):
    return pl.pallas_call(
        _tc_loss_body,
        out_shape=jax.ShapeDtypeStruct((1, 1), jnp.float32),
        out_specs=pl.BlockSpec(memory_space=pltpu.SMEM),
    )(scores2d)


def kernel(block_outputs, pos_edge_index, neg_edge_index):
    src = jnp.concatenate([pos_edge_index[0], neg_edge_index[0]])
    dst = jnp.concatenate([pos_edge_index[1], neg_edge_index[1]])
    scores = _sc_scores(block_outputs, src, dst)
    loss = _tc_loss(scores.reshape(E // 128, 128))
    return loss[0, 0]
